# baseline (speedup 1.0000x reference)
import jax
import jax.numpy as jnp
from jax.experimental import pallas as pl
from jax.experimental.pallas import tpu as pltpu

B, D, H, K, L = 1024, 32, 64, 512, 3
K_TILE = 64
NK = K // K_TILE
B_TILE = 256
NB = B // B_TILE


def _enc_kernel(x_ref, bw_ref, bb_ref, w_ref, lb_ref,
                enc_ref, out_ref,
                cur_ref, u_ref, min_ref, idx_ref, delta_ref):
    i = pl.program_id(0)
    k = pl.program_id(1)
    b = pl.program_id(2)
    bs = pl.ds(b * B_TILE, B_TILE)

    @pl.when(jnp.logical_and(i == 0, k == 0))
    def _init():
        cur_ref[bs, :] = jnp.zeros((B_TILE, D), jnp.float32)

    @pl.when(k == 0)
    def _stage_start():
        cur = cur_ref[bs, :]
        u = jnp.dot(cur, bw_ref[...], preferred_element_type=jnp.float32)
        u = jnp.maximum(u + bb_ref[...], 0.0)
        u_ref[bs, :] = u
        min_ref[bs, :] = jnp.full((B_TILE, 1), jnp.inf, jnp.float32)

    cur = cur_ref[bs, :]
    mm = jnp.dot(u_ref[bs, :], w_ref[0], preferred_element_type=jnp.float32)
    delta = lb_ref[0][None, :, :] + mm.reshape(B_TILE, K_TILE, D)
    cand = cur[:, None, :] + delta
    diff = cand - x_ref[bs, :][:, None, :]
    losses = jnp.mean(diff * diff, axis=-1)

    tmin = jnp.min(losses, axis=-1, keepdims=True)
    targ = jnp.argmin(losses, axis=-1).astype(jnp.int32)[:, None]
    onehot3 = (jax.lax.broadcasted_iota(jnp.int32, (B_TILE, K_TILE, D), 1)
               == targ[:, :, None])
    tdelta = jnp.sum(jnp.where(onehot3, delta, 0.0), axis=1)

    upd = tmin < min_ref[bs, :]
    min_ref[bs, :] = jnp.where(upd, tmin, min_ref[bs, :])
    idx_ref[bs, :] = jnp.where(upd, targ + k * K_TILE, idx_ref[bs, :])
    delta_ref[bs, :] = jnp.where(upd, tdelta, delta_ref[bs, :])

    @pl.when(k == NK - 1)
    def _stage_end():
        enc_ref[i, bs, :] = idx_ref[bs, :]
        newcur = cur_ref[bs, :] + delta_ref[bs, :]
        cur_ref[bs, :] = newcur

        @pl.when(i == L - 1)
        def _done():
            out_ref[bs, :] = newcur


def kernel(inputs, base_W, base_b, layer_Ws, layer_biases):
    enc, cur = pl.pallas_call(
        _enc_kernel,
        grid=(L, NK, NB),
        in_specs=[
            pl.BlockSpec((B, D), lambda i, k, b: (0, 0)),
            pl.BlockSpec((D, H), lambda i, k, b: (0, 0)),
            pl.BlockSpec((1, H), lambda i, k, b: (0, 0)),
            pl.BlockSpec((1, H, K_TILE * D), lambda i, k, b: (i, 0, k)),
            pl.BlockSpec((1, K_TILE, D), lambda i, k, b: (i, k, 0)),
        ],
        out_specs=[
            pl.BlockSpec((L, B, 1), lambda i, k, b: (0, 0, 0)),
            pl.BlockSpec((B, D), lambda i, k, b: (0, 0)),
        ],
        out_shape=[
            jax.ShapeDtypeStruct((L, B, 1), jnp.int32),
            jax.ShapeDtypeStruct((B, D), jnp.float32),
        ],
        scratch_shapes=[
            pltpu.VMEM((B, D), jnp.float32),
            pltpu.VMEM((B, H), jnp.float32),
            pltpu.VMEM((B, 1), jnp.float32),
            pltpu.VMEM((B, 1), jnp.int32),
            pltpu.VMEM((B, D), jnp.float32),
        ],
    )(inputs, base_W, base_b.reshape(1, H), layer_Ws, layer_biases)
    return enc[:, :, 0].T, cur


# speedup vs baseline: 4.9794x; 4.9794x over previous
import jax
import jax.numpy as jnp
from jax.experimental import pallas as pl
from jax.experimental.pallas import tpu as pltpu

B, D, H, K, L = 1024, 32, 64, 512, 3
B_TILE = 256
NB = B // B_TILE


def _enc_kernel(x_ref, bw_ref, bb_ref, w_ref, lb_ref,
                enc_ref, out_ref, cur_ref, delta_ref):
    i = pl.program_id(0)
    b = pl.program_id(1)
    bs = pl.ds(b * B_TILE, B_TILE)

    @pl.when(i == 0)
    def _init():
        cur_ref[bs, :] = jnp.zeros((B_TILE, D), jnp.float32)

    cur = cur_ref[bs, :]
    u = jnp.dot(cur, bw_ref[...], preferred_element_type=jnp.float32)
    u = jnp.maximum(u + bb_ref[...], 0.0)

    def sq_d(d):
        mm = jnp.dot(u, w_ref[0, d], preferred_element_type=jnp.float32)
        ld = lb_ref[0, d] + mm
        delta_ref[d, :, :] = ld
        cd = cur[:, d:d + 1] + ld
        fd = cd - x_ref[bs, d:d + 1]
        return fd * fd

    level = [sq_d(d) + sq_d(d + 16) for d in range(16)]
    while len(level) > 1:
        half = len(level) // 2
        level = [level[j] + level[j + half] for j in range(half)]
    losses = level[0] * jnp.float32(1.0 / D)

    targ = jnp.argmin(losses, axis=-1).astype(jnp.int32)[:, None]
    enc_ref[i, bs, :] = targ
    mask = jax.lax.broadcasted_iota(jnp.int32, (B_TILE, K), 1) == targ
    cols = [jnp.sum(jnp.where(mask, delta_ref[d, :, :], 0.0),
                    axis=1, keepdims=True) for d in range(D)]
    newcur = cur + jnp.concatenate(cols, axis=1)
    cur_ref[bs, :] = newcur

    @pl.when(i == L - 1)
    def _done():
        out_ref[bs, :] = newcur


def kernel(inputs, base_W, base_b, layer_Ws, layer_biases):
    wd = layer_Ws.reshape(L, H, K, D).transpose(0, 3, 1, 2)
    lbd = layer_biases.transpose(0, 2, 1)[:, :, None, :]
    enc, cur = pl.pallas_call(
        _enc_kernel,
        grid=(L, NB),
        in_specs=[
            pl.BlockSpec((B, D), lambda i, b: (0, 0)),
            pl.BlockSpec((D, H), lambda i, b: (0, 0)),
            pl.BlockSpec((1, H), lambda i, b: (0, 0)),
            pl.BlockSpec((1, D, H, K), lambda i, b: (i, 0, 0, 0)),
            pl.BlockSpec((1, D, 1, K), lambda i, b: (i, 0, 0, 0)),
        ],
        out_specs=[
            pl.BlockSpec((L, B, 1), lambda i, b: (0, 0, 0)),
            pl.BlockSpec((B, D), lambda i, b: (0, 0)),
        ],
        out_shape=[
            jax.ShapeDtypeStruct((L, B, 1), jnp.int32),
            jax.ShapeDtypeStruct((B, D), jnp.float32),
        ],
        scratch_shapes=[
            pltpu.VMEM((B, D), jnp.float32),
            pltpu.VMEM((D, B_TILE, K), jnp.float32),
        ],
    )(inputs, base_W, base_b.reshape(1, H), wd, lbd)
    return enc[:, :, 0].T, cur
